# baseline (speedup 1.0000x reference)
import jax
import jax.numpy as jnp
from jax.experimental import pallas as pl
from jax.experimental.pallas import tpu as pltpu

_BM = 400
_R = 3


def _make_body(n, f, h, c, nstripes):
    def body(x_ref, a_hbm, w1_ref, b1_ref, w2_ref, b2_ref, out_hbm,
             *scr):
        rings = scr[:_R]
        s2_ref = scr[_R]
        obufs = scr[_R + 1:_R + 3]
        rsems = scr[_R + 3:3 * _R + 3]
        osems = scr[3 * _R + 3:3 * _R + 5]
        half = _BM // 2

        class _Pair:
            def __init__(self, i, slot):
                self.c1 = pltpu.make_async_copy(
                    a_hbm.at[pl.ds(i * _BM, half), :],
                    rings[slot].at[pl.ds(0, half), :], rsems[2 * slot])
                self.c2 = pltpu.make_async_copy(
                    a_hbm.at[pl.ds(i * _BM + half, half), :],
                    rings[slot].at[pl.ds(half, half), :], rsems[2 * slot + 1])

            def start(self):
                self.c1.start()
                self.c2.start()

            def wait(self):
                self.c1.wait()
                self.c2.wait()

        def stripe_copy(i, slot):
            return _Pair(i, slot)

        def out_copy(j, buf):
            return pltpu.make_async_copy(
                obufs[buf], out_hbm.at[pl.ds(j * _BM, _BM), :], osems[buf])

        for i in range(_R):
            stripe_copy(i, i).start()

        for i in range(nstripes):
            slot = i % _R
            stripe_copy(i, slot).wait()
            ax = jnp.dot(rings[slot][...], x_ref[...])
            hh = jnp.dot(ax, w1_ref[...]) + b1_ref[...].reshape(1, h)
            s2_ref[pl.ds(i * _BM, _BM), :] = jnp.dot(
                jnp.maximum(hh, 0.0), w2_ref[...])
            if i + _R < nstripes:
                stripe_copy(i + _R, slot).start()

        out_js = list(range(nstripes - 1, -1, -1))
        for idx, j in enumerate(out_js):
            slot = j % _R
            if j < nstripes - _R:
                stripe_copy(j, slot).wait()
            o = jnp.dot(rings[slot][...], s2_ref[...]) + b2_ref[...].reshape(1, c)
            m = jnp.max(o, axis=1, keepdims=True)
            e = jnp.exp(o - m)
            res = e / jnp.sum(e, axis=1, keepdims=True)
            buf = idx % 2
            if idx >= 2:
                out_copy(out_js[idx - 2], buf).wait()
            obufs[buf][...] = res
            out_copy(j, buf).start()
            if j - _R >= 0:
                stripe_copy(j - _R, slot).start()

        out_copy(out_js[-2], (nstripes - 2) % 2).wait()
        out_copy(out_js[-1], (nstripes - 1) % 2).wait()

    return body


def kernel(in_feat, adj_mat, W1, b1, W2, b2):
    n, f = in_feat.shape
    h = W1.shape[1]
    c = W2.shape[1]
    nstripes = n // _BM

    vmem = pltpu.MemorySpace.VMEM
    return pl.pallas_call(
        _make_body(n, f, h, c, nstripes),
        in_specs=[
            pl.BlockSpec(memory_space=vmem),
            pl.BlockSpec(memory_space=pl.ANY),
            pl.BlockSpec(memory_space=vmem),
            pl.BlockSpec(memory_space=vmem),
            pl.BlockSpec(memory_space=vmem),
            pl.BlockSpec(memory_space=vmem),
        ],
        out_specs=pl.BlockSpec(memory_space=pl.ANY),
        out_shape=jax.ShapeDtypeStruct((n, c), jnp.float32),
        scratch_shapes=(
            [pltpu.VMEM((_BM, n), jnp.float32) for _ in range(_R)]
            + [pltpu.VMEM((n, c), jnp.float32)]
            + [pltpu.VMEM((_BM, c), jnp.float32) for _ in range(2)]
            + [pltpu.SemaphoreType.DMA for _ in range(2 * _R + 2)]
        ),
        compiler_params=pltpu.CompilerParams(
            vmem_limit_bytes=128 * 1024 * 1024,
        ),
    )(in_feat, adj_mat, W1, b1, W2, b2)


# speedup vs baseline: 1.0314x; 1.0314x over previous
import jax
import jax.numpy as jnp
from jax.experimental import pallas as pl
from jax.experimental.pallas import tpu as pltpu

_BM = 400
_R = 3
_GRP = 8


def _make_body(n, f, h, c, nstripes):
    def body(x_ref, a_hbm, w1_ref, b1_ref, w2t_ref, b2_ref, out_hbm, *scr):
        rings = scr[:_R]
        s2_ref = scr[_R]
        w2_ref = scr[_R + 1]
        obuf = scr[_R + 2]
        rsems = scr[_R + 3:3 * _R + 3]
        osem = scr[3 * _R + 3]
        half = _BM // 2

        class _Pair:
            def __init__(self, i, slot):
                self.c1 = pltpu.make_async_copy(
                    a_hbm.at[pl.ds(i * _BM, half), :],
                    rings[slot].at[pl.ds(0, half), :], rsems[2 * slot])
                self.c2 = pltpu.make_async_copy(
                    a_hbm.at[pl.ds(i * _BM + half, half), :],
                    rings[slot].at[pl.ds(half, half), :], rsems[2 * slot + 1])

            def start(self):
                self.c1.start()
                self.c2.start()

            def wait(self):
                self.c1.wait()
                self.c2.wait()

        def stripe_copy(i, slot):
            return _Pair(i, slot)

        for i in range(_R):
            stripe_copy(i, i).start()

        w2_ref[...] = w2t_ref[...].T

        for i in range(nstripes):
            slot = i % _R
            stripe_copy(i, slot).wait()
            ax = jnp.dot(rings[slot][...], x_ref[...])
            hh = jnp.dot(ax, w1_ref[...]) + b1_ref[...].reshape(1, h)
            s2_ref[pl.ds(i * _BM, _BM), :] = jnp.dot(
                jnp.maximum(hh, 0.0), w2_ref[...])
            if i + _R < nstripes:
                stripe_copy(i + _R, slot).start()

        for j in range(nstripes - 1, -1, -1):
            slot = j % _R
            if j < nstripes - _R:
                stripe_copy(j, slot).wait()
            o = jnp.dot(rings[slot][...], s2_ref[...]) \
                + b2_ref[...].reshape(1, c)
            m = jnp.max(o, axis=1, keepdims=True)
            e = jnp.exp(o - m)
            res = e / jnp.sum(e, axis=1, keepdims=True)
            obuf[:, pl.ds(j * _BM, _BM)] = res.T
            if j - _R >= 0:
                stripe_copy(j - _R, slot).start()
        pltpu.make_async_copy(obuf, out_hbm, osem).start()
        pltpu.make_async_copy(obuf, out_hbm, osem).wait()

    return body


def kernel(in_feat, adj_mat, W1, b1, W2, b2):
    n, f = in_feat.shape
    h = W1.shape[1]
    c = W2.shape[1]
    nstripes = n // _BM

    vmem = pltpu.MemorySpace.VMEM
    outt = pl.pallas_call(
        _make_body(n, f, h, c, nstripes),
        in_specs=[
            pl.BlockSpec(memory_space=vmem),
            pl.BlockSpec(memory_space=pl.ANY),
            pl.BlockSpec(memory_space=vmem),
            pl.BlockSpec(memory_space=vmem),
            pl.BlockSpec(memory_space=vmem),
            pl.BlockSpec(memory_space=vmem),
        ],
        out_specs=pl.BlockSpec(memory_space=pl.ANY),
        out_shape=jax.ShapeDtypeStruct((c, n), jnp.float32),
        scratch_shapes=(
            [pltpu.VMEM((_BM, n), jnp.float32) for _ in range(_R)]
            + [pltpu.VMEM((n, c), jnp.float32),
               pltpu.VMEM((h, c), jnp.float32),
               pltpu.VMEM((c, n), jnp.float32)]
            + [pltpu.SemaphoreType.DMA for _ in range(2 * _R + 1)]
        ),
        compiler_params=pltpu.CompilerParams(
            vmem_limit_bytes=128 * 1024 * 1024,
        ),
    )(in_feat, adj_mat, W1, b1, W2.T, b2)
    return outt.T
